# baseline (speedup 1.0000x reference)
import jax
import jax.numpy as jnp
from jax import lax
from jax.experimental import pallas as pl
from jax.experimental.pallas import tpu as pltpu
from jax.experimental.pallas import tpu_sc as plsc

NUM_CLASSES = 150
FEAT = 256
B = 8
H = 128
W = 128
NC, NS, L = 2, 16, 16
NW = NC * NS
RPW = (B * H) // NW
PXW = RPW * W
GPR = W // L
KCH = 8
NCHUNK = FEAT // KCH
SUMS = FEAT * NUM_CLASSES
CPAD = 160


def _worker(axis_c, axis_s):
    wid = lax.axis_index(axis_s) * NC + lax.axis_index(axis_c)
    b = wid // (NW // B)
    h0 = (wid % (NW // B)) * RPW
    return wid, b, h0


def _proto_body(t_ref, m_ref, z_ref, sums_out, counts_out,
                m_v, buf0, bank0, bank1, counts_v, sem0):
    wid, b, h0 = _worker("c", "s")
    pltpu.sync_copy(z_ref, bank0)
    pltpu.sync_copy(z_ref, bank1)
    for k in range(CPAD // L):
        counts_v[pl.ds(k * L, L)] = jnp.zeros((L,), jnp.float32)
    pltpu.sync_copy(m_ref.at[b, pl.ds(h0, RPW)], m_v)

    def copy(c0, sem):
        return pltpu.make_async_copy(
            t_ref.at[b, pl.ds(c0 * KCH, KCH), pl.ds(h0, RPW)], buf0, sem)

    copy(0, sem0).start()

    ones = jnp.ones((L,), jnp.float32)

    def cnt_body(r, _):
        for u in range(GPR):
            idx = m_v[r, pl.ds(u * L, L)]
            plsc.addupdate_scatter(counts_v, [idx], ones)
        return 0
    lax.fori_loop(0, RPW, cnt_body, 0)

    def chunk_body(c0, _):
        copy(c0, sem0).wait()

        def r_body(r, _):
            for u in range(GPR):
                idx = m_v[r, pl.ds(u * L, L)]
                for ch in range(KCH):
                    x = buf0[ch, r, pl.ds(u * L, L)]
                    bank = bank0 if ch % 2 == 0 else bank1
                    plsc.addupdate_scatter(
                        bank, [idx + (c0 * KCH + ch) * NUM_CLASSES], x)
            return 0
        lax.fori_loop(0, RPW, r_body, 0)

        @pl.when(c0 < NCHUNK - 1)
        def _():
            copy(c0 + 1, sem0).start()
        return 0
    lax.fori_loop(0, NCHUNK, chunk_body, 0)

    def merge_body(k, _):
        off = k * L
        bank0[pl.ds(off, L)] = bank0[pl.ds(off, L)] + bank1[pl.ds(off, L)]
        return 0
    lax.fori_loop(0, SUMS // L, merge_body, 0)

    pltpu.sync_copy(bank0, sums_out.at[wid])
    pltpu.sync_copy(counts_v, counts_out.at[wid])


def _pixel_body(s_ref, m_ref, w_ref, pt_ref, pv_ref, z_ref,
                sim_out, wl_out, part_out,
                m_v, w_v, buf0, buf1, pt_v, pv_v, nsq_v, dot_v, sim_v, wl_v,
                acc_v, sem0, sem1):
    wid, b, h0 = _worker("c", "s")

    def copy(c0, buf, sem):
        return pltpu.make_async_copy(
            s_ref.at[b, pl.ds(c0 * KCH, KCH), pl.ds(h0, RPW)], buf, sem)

    copy(0, buf0, sem0).start()
    pltpu.sync_copy(m_ref.at[b, pl.ds(h0, RPW)], m_v)
    pltpu.sync_copy(w_ref.at[b, pl.ds(h0, RPW)], w_v)
    pltpu.sync_copy(pt_ref, pt_v)
    pltpu.sync_copy(pv_ref, pv_v)
    pltpu.sync_copy(z_ref, nsq_v)
    pltpu.sync_copy(z_ref, dot_v)
    for k in range(4):
        acc_v[pl.ds(k * L, L)] = jnp.zeros((L,), jnp.float32)

    def compute(c0, buf):
        def r_body(r, _):
            for u in range(GPR):
                off = r * W + u * L
                idx = m_v[r, pl.ds(u * L, L)]
                nsq = nsq_v[pl.ds(off, L)]
                dot = dot_v[pl.ds(off, L)]
                for jj in range(KCH // 2):
                    pv = plsc.load_gather(
                        pt_v, [idx + (c0 * KCH // 2 + jj) * NUM_CLASSES])
                    p0, p1 = plsc.unpack(
                        plsc.bitcast(pv, jnp.bfloat16),
                        format=plsc.PackFormat.INTERLEAVED)
                    x0 = buf[2 * jj, r, pl.ds(u * L, L)]
                    x1 = buf[2 * jj + 1, r, pl.ds(u * L, L)]
                    nsq = nsq + x0 * x0 + x1 * x1
                    dot = dot + x0 * p0 + x1 * p1
                nsq_v[pl.ds(off, L)] = nsq
                dot_v[pl.ds(off, L)] = dot
            return 0
        lax.fori_loop(0, RPW, r_body, 0)

    def pair_body(i, _):
        ca = 2 * i
        cb = 2 * i + 1
        copy(ca, buf0, sem0).wait()
        copy(cb, buf1, sem1).start()
        compute(ca, buf0)
        copy(cb, buf1, sem1).wait()

        @pl.when(i < NCHUNK // 2 - 1)
        def _():
            copy(ca + 2, buf0, sem0).start()
        compute(cb, buf1)
        return 0
    lax.fori_loop(0, NCHUNK // 2, pair_body, 0)

    def fin_body(r, _):
        for u in range(GPR):
            off = r * W + u * L
            x = nsq_v[pl.ds(off, L)]
            d = dot_v[pl.ds(off, L)]
            m16 = m_v[r, pl.ds(u * L, L)]
            sw = w_v[r, pl.ds(u * L, L)]
            i = plsc.bitcast(x, jnp.int32)
            i = jnp.int32(0x5F3759DF) - (i >> 1)
            y = plsc.bitcast(i, jnp.float32)
            for _ in range(3):
                y = y * (1.5 - 0.5 * x * y * y)
            nrm = x * y
            sim = jnp.where(nrm > 1e-12, d * y, d * 1e12)
            v = plsc.load_gather(pv_v, [m16])
            wl = (1.0 - sim) * v * sw
            sim_v[r, pl.ds(u * L, L)] = sim
            wl_v[r, pl.ds(u * L, L)] = wl
            acc_v[pl.ds(0, L)] = acc_v[pl.ds(0, L)] + wl
            acc_v[pl.ds(L, L)] = acc_v[pl.ds(L, L)] + v * sw
            acc_v[pl.ds(2 * L, L)] = acc_v[pl.ds(2 * L, L)] + sim * v
            acc_v[pl.ds(3 * L, L)] = acc_v[pl.ds(3 * L, L)] + v
        return 0
    lax.fori_loop(0, RPW, fin_body, 0)

    pltpu.sync_copy(sim_v, sim_out.at[b, pl.ds(h0, RPW)])
    pltpu.sync_copy(wl_v, wl_out.at[b, pl.ds(h0, RPW)])
    pltpu.sync_copy(acc_v, part_out.at[wid])


def _sc_mesh():
    return plsc.VectorSubcoreMesh(core_axis_name="c", subcore_axis_name="s",
                                  num_cores=NC, num_subcores=NS)


_SC_PARAMS = pltpu.CompilerParams(needs_layout_passes=False)


@jax.jit
def kernel(s_feat, t_feat, mask, swt_weight):
    f32 = jnp.float32
    w3 = swt_weight.reshape(B, H, W)
    zeros = jnp.zeros((SUMS,), f32)
    zeros_px = jnp.zeros((PXW,), f32)

    proto_call = pl.kernel(
        _proto_body,
        out_type=(jax.ShapeDtypeStruct((NW, SUMS), f32),
                  jax.ShapeDtypeStruct((NW, CPAD), f32)),
        mesh=_sc_mesh(),
        compiler_params=_SC_PARAMS,
        scratch_types=[
            pltpu.VMEM((RPW, W), jnp.int32),
            pltpu.VMEM((KCH, RPW, W), f32),
            pltpu.VMEM((SUMS,), f32),
            pltpu.VMEM((SUMS,), f32),
            pltpu.VMEM((CPAD,), f32),
            pltpu.SemaphoreType.DMA,
        ],
    )
    sums_part, counts_part = proto_call(t_feat, mask, zeros)
    sums = sums_part.sum(0).reshape(FEAT, NUM_CLASSES).T
    counts = counts_part.sum(0)[:NUM_CLASSES]
    means = jnp.where(counts[:, None] > 0,
                      sums / jnp.maximum(counts[:, None], 1.0), 0.0)
    nrm = jnp.linalg.norm(means, axis=-1, keepdims=True)
    protos = means / jnp.maximum(nrm, 1e-6)
    proto_norm = jnp.linalg.norm(protos, axis=-1)
    proto_valid = proto_norm > 0
    pt_pack = jax.lax.bitcast_convert_type(
        protos.astype(jnp.bfloat16).reshape(NUM_CLASSES, FEAT // 2, 2),
        jnp.float32).T.reshape(-1)
    pv_f = jnp.zeros((CPAD,), f32).at[:NUM_CLASSES].set(proto_valid.astype(f32))

    pixel_call = pl.kernel(
        _pixel_body,
        out_type=(jax.ShapeDtypeStruct((B, H, W), f32),
                  jax.ShapeDtypeStruct((B, H, W), f32),
                  jax.ShapeDtypeStruct((NW, 4 * L), f32)),
        mesh=_sc_mesh(),
        compiler_params=_SC_PARAMS,
        scratch_types=[
            pltpu.VMEM((RPW, W), jnp.int32),
            pltpu.VMEM((RPW, W), f32),
            pltpu.VMEM((KCH, RPW, W), f32),
            pltpu.VMEM((KCH, RPW, W), f32),
            pltpu.VMEM((SUMS // 2,), f32),
            pltpu.VMEM((CPAD,), f32),
            pltpu.VMEM((PXW,), f32),
            pltpu.VMEM((PXW,), f32),
            pltpu.VMEM((RPW, W), f32),
            pltpu.VMEM((RPW, W), f32),
            pltpu.VMEM((4 * L,), f32),
            pltpu.SemaphoreType.DMA,
            pltpu.SemaphoreType.DMA,
        ],
    )
    sim_map, weighted_loss_map, partials = pixel_call(
        s_feat, mask, w3, pt_pack, pv_f, zeros_px)

    p = partials.reshape(NW, 4, L).sum(axis=(0, 2))
    final_loss = p[0] / jnp.maximum(p[1], 1.0)
    mean_sim = p[2] / jnp.maximum(p[3], 1.0)
    return (final_loss, sim_map, weighted_loss_map, mean_sim, proto_norm, proto_valid)


# speedup vs baseline: 1.1436x; 1.1436x over previous
import jax
import jax.numpy as jnp
from jax import lax
from jax.experimental import pallas as pl
from jax.experimental.pallas import tpu as pltpu
from jax.experimental.pallas import tpu_sc as plsc

NUM_CLASSES = 150
FEAT = 256
B = 8
H = 128
W = 128
NC, NS, L = 2, 16, 16
NW = NC * NS
RPW = (B * H) // NW
PXW = RPW * W
GPR = W // L
KCH = 8
NCHUNK = FEAT // KCH
SUMS = FEAT * NUM_CLASSES
CPAD = 160


def _worker(axis_c, axis_s):
    wid = lax.axis_index(axis_s) * NC + lax.axis_index(axis_c)
    b = wid // (NW // B)
    h0 = (wid % (NW // B)) * RPW
    return wid, b, h0


def _proto_body(t_ref, m_ref, z_ref, sums_out, counts_out,
                m_v, buf0, buf1, sums_v, counts_v, sem0, sem1):
    wid, b, h0 = _worker("c", "s")
    pltpu.sync_copy(z_ref, sums_v)
    for k in range(CPAD // L):
        counts_v[pl.ds(k * L, L)] = jnp.zeros((L,), jnp.float32)
    pltpu.sync_copy(m_ref.at[b, pl.ds(h0, RPW)], m_v)

    def copy(c0, buf, sem):
        return pltpu.make_async_copy(
            t_ref.at[b, pl.ds(c0 * KCH, KCH), pl.ds(h0, RPW)], buf, sem)

    copy(0, buf0, sem0).start()

    ones = jnp.ones((L,), jnp.float32)

    def cnt_body(r, _):
        for u in range(GPR):
            idx = m_v[r, pl.ds(u * L, L)]
            plsc.addupdate_scatter(counts_v, [idx], ones)
        return 0
    lax.fori_loop(0, RPW, cnt_body, 0)

    def compute(c0, buf):
        def r_body(r, _):
            for u in range(GPR):
                idx = m_v[r, pl.ds(u * L, L)]
                for ch in range(KCH):
                    x = buf[ch, r, pl.ds(u * L, L)]
                    plsc.addupdate_scatter(
                        sums_v, [idx + (c0 * KCH + ch) * NUM_CLASSES], x)
            return 0
        lax.fori_loop(0, RPW, r_body, 0)

    def pair_body(i, _):
        ca = 2 * i
        cb = 2 * i + 1
        copy(ca, buf0, sem0).wait()
        copy(cb, buf1, sem1).start()
        compute(ca, buf0)
        copy(cb, buf1, sem1).wait()

        @pl.when(i < NCHUNK // 2 - 1)
        def _():
            copy(ca + 2, buf0, sem0).start()
        compute(cb, buf1)
        return 0
    lax.fori_loop(0, NCHUNK // 2, pair_body, 0)

    pltpu.sync_copy(sums_v, sums_out.at[wid])
    pltpu.sync_copy(counts_v, counts_out.at[wid])


def _pixel_body(s_ref, m_ref, w_ref, pt_ref, pv_ref, z_ref,
                sim_out, wl_out, part_out,
                m_v, w_v, buf0, buf1, pt_v, pv_v, nsq_v, dot_v, sim_v, wl_v,
                acc_v, sem0, sem1):
    wid, b, h0 = _worker("c", "s")

    def copy(c0, buf, sem):
        return pltpu.make_async_copy(
            s_ref.at[b, pl.ds(c0 * KCH, KCH), pl.ds(h0, RPW)], buf, sem)

    copy(0, buf0, sem0).start()
    pltpu.sync_copy(m_ref.at[b, pl.ds(h0, RPW)], m_v)
    pltpu.sync_copy(w_ref.at[b, pl.ds(h0, RPW)], w_v)
    pltpu.sync_copy(pt_ref, pt_v)
    pltpu.sync_copy(pv_ref, pv_v)
    pltpu.sync_copy(z_ref, nsq_v)
    pltpu.sync_copy(z_ref, dot_v)
    for k in range(4):
        acc_v[pl.ds(k * L, L)] = jnp.zeros((L,), jnp.float32)

    def compute(c0, buf):
        def r_body(r, _):
            for u in range(GPR):
                off = r * W + u * L
                idx = m_v[r, pl.ds(u * L, L)]
                nsq = nsq_v[pl.ds(off, L)]
                dot = dot_v[pl.ds(off, L)]
                for ch in range(KCH):
                    x = buf[ch, r, pl.ds(u * L, L)]
                    p = plsc.load_gather(
                        pt_v, [idx + (c0 * KCH + ch) * NUM_CLASSES])
                    nsq = nsq + x * x
                    dot = dot + x * p
                nsq_v[pl.ds(off, L)] = nsq
                dot_v[pl.ds(off, L)] = dot
            return 0
        lax.fori_loop(0, RPW, r_body, 0)

    def pair_body(i, _):
        ca = 2 * i
        cb = 2 * i + 1
        copy(ca, buf0, sem0).wait()
        copy(cb, buf1, sem1).start()
        compute(ca, buf0)
        copy(cb, buf1, sem1).wait()

        @pl.when(i < NCHUNK // 2 - 1)
        def _():
            copy(ca + 2, buf0, sem0).start()
        compute(cb, buf1)
        return 0
    lax.fori_loop(0, NCHUNK // 2, pair_body, 0)

    def fin_body(r, _):
        for u in range(GPR):
            off = r * W + u * L
            x = nsq_v[pl.ds(off, L)]
            d = dot_v[pl.ds(off, L)]
            m16 = m_v[r, pl.ds(u * L, L)]
            sw = w_v[r, pl.ds(u * L, L)]
            i = plsc.bitcast(x, jnp.int32)
            i = jnp.int32(0x5F3759DF) - (i >> 1)
            y = plsc.bitcast(i, jnp.float32)
            for _ in range(3):
                y = y * (1.5 - 0.5 * x * y * y)
            nrm = x * y
            sim = jnp.where(nrm > 1e-12, d * y, d * 1e12)
            v = plsc.load_gather(pv_v, [m16])
            wl = (1.0 - sim) * v * sw
            sim_v[r, pl.ds(u * L, L)] = sim
            wl_v[r, pl.ds(u * L, L)] = wl
            acc_v[pl.ds(0, L)] = acc_v[pl.ds(0, L)] + wl
            acc_v[pl.ds(L, L)] = acc_v[pl.ds(L, L)] + v * sw
            acc_v[pl.ds(2 * L, L)] = acc_v[pl.ds(2 * L, L)] + sim * v
            acc_v[pl.ds(3 * L, L)] = acc_v[pl.ds(3 * L, L)] + v
        return 0
    lax.fori_loop(0, RPW, fin_body, 0)

    pltpu.sync_copy(sim_v, sim_out.at[b, pl.ds(h0, RPW)])
    pltpu.sync_copy(wl_v, wl_out.at[b, pl.ds(h0, RPW)])
    pltpu.sync_copy(acc_v, part_out.at[wid])


def _sc_mesh():
    return plsc.VectorSubcoreMesh(core_axis_name="c", subcore_axis_name="s",
                                  num_cores=NC, num_subcores=NS)


_SC_PARAMS = pltpu.CompilerParams(needs_layout_passes=False)


@jax.jit
def kernel(s_feat, t_feat, mask, swt_weight):
    f32 = jnp.float32
    w3 = swt_weight.reshape(B, H, W)
    zeros = jnp.zeros((SUMS,), f32)
    zeros_px = jnp.zeros((PXW,), f32)

    proto_call = pl.kernel(
        _proto_body,
        out_type=(jax.ShapeDtypeStruct((NW, SUMS), f32),
                  jax.ShapeDtypeStruct((NW, CPAD), f32)),
        mesh=_sc_mesh(),
        compiler_params=_SC_PARAMS,
        scratch_types=[
            pltpu.VMEM((RPW, W), jnp.int32),
            pltpu.VMEM((KCH, RPW, W), f32),
            pltpu.VMEM((KCH, RPW, W), f32),
            pltpu.VMEM((SUMS,), f32),
            pltpu.VMEM((CPAD,), f32),
            pltpu.SemaphoreType.DMA,
            pltpu.SemaphoreType.DMA,
        ],
    )
    sums_part, counts_part = proto_call(t_feat, mask, zeros)
    sums = sums_part.sum(0).reshape(FEAT, NUM_CLASSES).T
    counts = counts_part.sum(0)[:NUM_CLASSES]
    means = jnp.where(counts[:, None] > 0,
                      sums / jnp.maximum(counts[:, None], 1.0), 0.0)
    nrm = jnp.linalg.norm(means, axis=-1, keepdims=True)
    protos = means / jnp.maximum(nrm, 1e-6)
    proto_norm = jnp.linalg.norm(protos, axis=-1)
    proto_valid = proto_norm > 0
    pt_flat = protos.T.reshape(-1)
    pv_f = jnp.zeros((CPAD,), f32).at[:NUM_CLASSES].set(proto_valid.astype(f32))

    pixel_call = pl.kernel(
        _pixel_body,
        out_type=(jax.ShapeDtypeStruct((B, H, W), f32),
                  jax.ShapeDtypeStruct((B, H, W), f32),
                  jax.ShapeDtypeStruct((NW, 4 * L), f32)),
        mesh=_sc_mesh(),
        compiler_params=_SC_PARAMS,
        scratch_types=[
            pltpu.VMEM((RPW, W), jnp.int32),
            pltpu.VMEM((RPW, W), f32),
            pltpu.VMEM((KCH, RPW, W), f32),
            pltpu.VMEM((KCH, RPW, W), f32),
            pltpu.VMEM((SUMS,), f32),
            pltpu.VMEM((CPAD,), f32),
            pltpu.VMEM((PXW,), f32),
            pltpu.VMEM((PXW,), f32),
            pltpu.VMEM((RPW, W), f32),
            pltpu.VMEM((RPW, W), f32),
            pltpu.VMEM((4 * L,), f32),
            pltpu.SemaphoreType.DMA,
            pltpu.SemaphoreType.DMA,
        ],
    )
    sim_map, weighted_loss_map, partials = pixel_call(
        s_feat, mask, w3, pt_flat, pv_f, zeros_px)

    p = partials.reshape(NW, 4, L).sum(axis=(0, 2))
    final_loss = p[0] / jnp.maximum(p[1], 1.0)
    mean_sim = p[2] / jnp.maximum(p[3], 1.0)
    return (final_loss, sim_map, weighted_loss_map, mean_sim, proto_norm, proto_valid)


# speedup vs baseline: 1.6003x; 1.3994x over previous
import jax
import jax.numpy as jnp
from jax import lax
from jax.experimental import pallas as pl
from jax.experimental.pallas import tpu as pltpu
from jax.experimental.pallas import tpu_sc as plsc

NUM_CLASSES = 150
FEAT = 256
B = 8
H = 128
W = 128
NC, NS, L = 2, 16, 16
NW = NC * NS
RPW = (B * H) // NW
PXW = RPW * W
GPR = W // L
KCH = 8
NCHUNK = FEAT // KCH
SUMS = FEAT * NUM_CLASSES
CPAD = 160


def _worker(axis_c, axis_s):
    wid = lax.axis_index(axis_s) * NC + lax.axis_index(axis_c)
    b = wid // (NW // B)
    h0 = (wid % (NW // B)) * RPW
    return wid, b, h0


def _proto_body(t_ref, m_ref, z_ref, sums_out, counts_out,
                m_v, buf0, buf1, sums_v, counts_v, sem0, sem1):
    wid, b, h0 = _worker("c", "s")
    pltpu.sync_copy(z_ref, sums_v)
    for k in range(CPAD // L):
        counts_v[pl.ds(k * L, L)] = jnp.zeros((L,), jnp.float32)
    pltpu.sync_copy(m_ref.at[b, pl.ds(h0, RPW)], m_v)

    def copy(c0, buf, sem):
        return pltpu.make_async_copy(
            t_ref.at[b, pl.ds(c0 * KCH, KCH), pl.ds(h0, RPW)], buf, sem)

    copy(0, buf0, sem0).start()

    ones = jnp.ones((L,), jnp.float32)

    def cnt_body(r, _):
        for u in range(GPR):
            idx = m_v[r, pl.ds(u * L, L)]
            plsc.addupdate_scatter(counts_v, [idx], ones)
        return 0
    lax.fori_loop(0, RPW, cnt_body, 0)

    def compute(c0, buf):
        def r_body(r, _):
            for u in range(GPR):
                idx = m_v[r, pl.ds(u * L, L)]
                for ch in range(KCH):
                    x = buf[ch, r, pl.ds(u * L, L)]
                    plsc.addupdate_scatter(
                        sums_v, [idx + (c0 * KCH + ch) * NUM_CLASSES], x)
            return 0
        lax.fori_loop(0, RPW, r_body, 0)

    def pair_body(i, _):
        ca = 2 * i
        cb = 2 * i + 1
        copy(ca, buf0, sem0).wait()
        copy(cb, buf1, sem1).start()
        compute(ca, buf0)
        copy(cb, buf1, sem1).wait()

        @pl.when(i < NCHUNK // 2 - 1)
        def _():
            copy(ca + 2, buf0, sem0).start()
        compute(cb, buf1)
        return 0
    lax.fori_loop(0, NCHUNK // 2, pair_body, 0)

    pltpu.sync_copy(sums_v, sums_out.at[wid])
    pltpu.sync_copy(counts_v, counts_out.at[wid])


def _pixel_body(s_ref, m_ref, w_ref, pt_ref, pv_ref, z_ref,
                sim_out, wl_out, part_out,
                m_v, w_v, buf0, buf1, pt_v, pv_v, nsq_v, dot_v, sim_v, wl_v,
                acc_v, sem0, sem1):
    wid, b, h0 = _worker("c", "s")

    def copy(c0, buf, sem):
        return pltpu.make_async_copy(
            s_ref.at[b, pl.ds(c0 * KCH, KCH), pl.ds(h0, RPW)], buf, sem)

    copy(0, buf0, sem0).start()
    pltpu.sync_copy(m_ref.at[b, pl.ds(h0, RPW)], m_v)
    pltpu.sync_copy(w_ref.at[b, pl.ds(h0, RPW)], w_v)
    pltpu.sync_copy(pt_ref, pt_v)
    pltpu.sync_copy(pv_ref, pv_v)
    pltpu.sync_copy(z_ref, nsq_v)
    pltpu.sync_copy(z_ref, dot_v)
    for k in range(4):
        acc_v[pl.ds(k * L, L)] = jnp.zeros((L,), jnp.float32)

    def compute(c0, buf):
        def r_body(r, _):
            for u in range(GPR):
                off = r * W + u * L
                idx = m_v[r, pl.ds(u * L, L)]
                nsq = nsq_v[pl.ds(off, L)]
                dot = dot_v[pl.ds(off, L)]
                for ch in range(KCH):
                    x = buf[ch, r, pl.ds(u * L, L)]
                    p = plsc.load_gather(
                        pt_v, [idx + (c0 * KCH + ch) * NUM_CLASSES])
                    nsq = nsq + x * x
                    dot = dot + x * p
                nsq_v[pl.ds(off, L)] = nsq
                dot_v[pl.ds(off, L)] = dot
            return 0
        lax.fori_loop(0, RPW, r_body, 0)

    def pair_body(i, _):
        ca = 2 * i
        cb = 2 * i + 1
        copy(ca, buf0, sem0).wait()
        copy(cb, buf1, sem1).start()
        compute(ca, buf0)
        copy(cb, buf1, sem1).wait()

        @pl.when(i < NCHUNK // 2 - 1)
        def _():
            copy(ca + 2, buf0, sem0).start()
        compute(cb, buf1)
        return 0
    lax.fori_loop(0, NCHUNK // 2, pair_body, 0)

    def fin_body(r, _):
        for u in range(GPR):
            off = r * W + u * L
            x = nsq_v[pl.ds(off, L)]
            d = dot_v[pl.ds(off, L)]
            m16 = m_v[r, pl.ds(u * L, L)]
            sw = w_v[r, pl.ds(u * L, L)]
            i = plsc.bitcast(x, jnp.int32)
            i = jnp.int32(0x5F3759DF) - (i >> 1)
            y = plsc.bitcast(i, jnp.float32)
            for _ in range(3):
                y = y * (1.5 - 0.5 * x * y * y)
            nrm = x * y
            sim = jnp.where(nrm > 1e-12, d * y, d * 1e12)
            v = plsc.load_gather(pv_v, [m16])
            wl = (1.0 - sim) * v * sw
            sim_v[r, pl.ds(u * L, L)] = sim
            wl_v[r, pl.ds(u * L, L)] = wl
            acc_v[pl.ds(0, L)] = acc_v[pl.ds(0, L)] + wl
            acc_v[pl.ds(L, L)] = acc_v[pl.ds(L, L)] + v * sw
            acc_v[pl.ds(2 * L, L)] = acc_v[pl.ds(2 * L, L)] + sim * v
            acc_v[pl.ds(3 * L, L)] = acc_v[pl.ds(3 * L, L)] + v
        return 0
    lax.fori_loop(0, RPW, fin_body, 0)

    pltpu.sync_copy(sim_v, sim_out.at[b, pl.ds(h0, RPW)])
    pltpu.sync_copy(wl_v, wl_out.at[b, pl.ds(h0, RPW)])
    pltpu.sync_copy(acc_v, part_out.at[wid])


HBLK = 8
CPAD_TC = 152


def _tc_proto_body(m_ref, t_ref, sums_ref, counts_ref):
    step = pl.program_id(0) * pl.num_programs(1) + pl.program_id(1)

    mrow = m_ref[...].reshape(1, HBLK * W)
    oh = (jax.lax.broadcasted_iota(jnp.int32, (CPAD_TC, HBLK * W), 0)
          == mrow).astype(jnp.float32)
    t = t_ref[...].reshape(FEAT, HBLK * W)
    part = jax.lax.dot_general(oh, t, (((1,), (1,)), ((), ())),
                               precision=jax.lax.Precision.HIGHEST,
                               preferred_element_type=jnp.float32)
    cnt = jnp.sum(oh, axis=1)

    @pl.when(step == 0)
    def _():
        sums_ref[...] = jnp.zeros_like(sums_ref)
        counts_ref[...] = jnp.zeros_like(counts_ref)
    sums_ref[...] += part
    counts_ref[...] += cnt


def _sc_mesh():
    return plsc.VectorSubcoreMesh(core_axis_name="c", subcore_axis_name="s",
                                  num_cores=NC, num_subcores=NS)


_SC_PARAMS = pltpu.CompilerParams(needs_layout_passes=False)


@jax.jit
def kernel(s_feat, t_feat, mask, swt_weight):
    f32 = jnp.float32
    w3 = swt_weight.reshape(B, H, W)
    zeros_px = jnp.zeros((PXW,), f32)

    sums_tc, counts_tc = pl.pallas_call(
        _tc_proto_body,
        grid=(B, H // HBLK),
        in_specs=[
            pl.BlockSpec((1, HBLK, W), lambda b, h: (b, h, 0)),
            pl.BlockSpec((1, FEAT, HBLK, W), lambda b, h: (b, 0, h, 0)),
        ],
        out_specs=[
            pl.BlockSpec((CPAD_TC, FEAT), lambda b, h: (0, 0)),
            pl.BlockSpec((1, CPAD_TC), lambda b, h: (0, 0)),
        ],
        out_shape=(jax.ShapeDtypeStruct((CPAD_TC, FEAT), f32),
                   jax.ShapeDtypeStruct((1, CPAD_TC), f32)),
    )(mask, t_feat)
    sums = sums_tc[:NUM_CLASSES]
    counts = counts_tc[0, :NUM_CLASSES]
    means = jnp.where(counts[:, None] > 0,
                      sums / jnp.maximum(counts[:, None], 1.0), 0.0)
    nrm = jnp.linalg.norm(means, axis=-1, keepdims=True)
    protos = means / jnp.maximum(nrm, 1e-6)
    proto_norm = jnp.linalg.norm(protos, axis=-1)
    proto_valid = proto_norm > 0
    pt_flat = protos.T.reshape(-1)
    pv_f = jnp.zeros((CPAD,), f32).at[:NUM_CLASSES].set(proto_valid.astype(f32))

    pixel_call = pl.kernel(
        _pixel_body,
        out_type=(jax.ShapeDtypeStruct((B, H, W), f32),
                  jax.ShapeDtypeStruct((B, H, W), f32),
                  jax.ShapeDtypeStruct((NW, 4 * L), f32)),
        mesh=_sc_mesh(),
        compiler_params=_SC_PARAMS,
        scratch_types=[
            pltpu.VMEM((RPW, W), jnp.int32),
            pltpu.VMEM((RPW, W), f32),
            pltpu.VMEM((KCH, RPW, W), f32),
            pltpu.VMEM((KCH, RPW, W), f32),
            pltpu.VMEM((SUMS,), f32),
            pltpu.VMEM((CPAD,), f32),
            pltpu.VMEM((PXW,), f32),
            pltpu.VMEM((PXW,), f32),
            pltpu.VMEM((RPW, W), f32),
            pltpu.VMEM((RPW, W), f32),
            pltpu.VMEM((4 * L,), f32),
            pltpu.SemaphoreType.DMA,
            pltpu.SemaphoreType.DMA,
        ],
    )
    sim_map, weighted_loss_map, partials = pixel_call(
        s_feat, mask, w3, pt_flat, pv_f, zeros_px)

    p = partials.reshape(NW, 4, L).sum(axis=(0, 2))
    final_loss = p[0] / jnp.maximum(p[1], 1.0)
    mean_sim = p[2] / jnp.maximum(p[3], 1.0)
    return (final_loss, sim_map, weighted_loss_map, mean_sim, proto_norm, proto_valid)


# speedup vs baseline: 1.6740x; 1.0461x over previous
import jax
import jax.numpy as jnp
from jax import lax
from jax.experimental import pallas as pl
from jax.experimental.pallas import tpu as pltpu
from jax.experimental.pallas import tpu_sc as plsc

NUM_CLASSES = 150
FEAT = 256
B = 8
H = 128
W = 128
NC, NS, L = 2, 16, 16
NW = NC * NS
RPW = (B * H) // NW
PXW = RPW * W
GPR = W // L
KCH = 8
NCHUNK = FEAT // KCH
SUMS = FEAT * NUM_CLASSES
CPAD = 160


def _worker(axis_c, axis_s):
    wid = lax.axis_index(axis_s) * NC + lax.axis_index(axis_c)
    b = wid // (NW // B)
    h0 = (wid % (NW // B)) * RPW
    return wid, b, h0


def _proto_body(t_ref, m_ref, z_ref, sums_out, counts_out,
                m_v, buf0, buf1, sums_v, counts_v, sem0, sem1):
    wid, b, h0 = _worker("c", "s")
    pltpu.sync_copy(z_ref, sums_v)
    for k in range(CPAD // L):
        counts_v[pl.ds(k * L, L)] = jnp.zeros((L,), jnp.float32)
    pltpu.sync_copy(m_ref.at[b, pl.ds(h0, RPW)], m_v)

    def copy(c0, buf, sem):
        return pltpu.make_async_copy(
            t_ref.at[b, pl.ds(c0 * KCH, KCH), pl.ds(h0, RPW)], buf, sem)

    copy(0, buf0, sem0).start()

    ones = jnp.ones((L,), jnp.float32)

    def cnt_body(r, _):
        for u in range(GPR):
            idx = m_v[r, pl.ds(u * L, L)]
            plsc.addupdate_scatter(counts_v, [idx], ones)
        return 0
    lax.fori_loop(0, RPW, cnt_body, 0)

    def compute(c0, buf):
        def r_body(r, _):
            for u in range(GPR):
                idx = m_v[r, pl.ds(u * L, L)]
                for ch in range(KCH):
                    x = buf[ch, r, pl.ds(u * L, L)]
                    plsc.addupdate_scatter(
                        sums_v, [idx + (c0 * KCH + ch) * NUM_CLASSES], x)
            return 0
        lax.fori_loop(0, RPW, r_body, 0)

    def pair_body(i, _):
        ca = 2 * i
        cb = 2 * i + 1
        copy(ca, buf0, sem0).wait()
        copy(cb, buf1, sem1).start()
        compute(ca, buf0)
        copy(cb, buf1, sem1).wait()

        @pl.when(i < NCHUNK // 2 - 1)
        def _():
            copy(ca + 2, buf0, sem0).start()
        compute(cb, buf1)
        return 0
    lax.fori_loop(0, NCHUNK // 2, pair_body, 0)

    pltpu.sync_copy(sums_v, sums_out.at[wid])
    pltpu.sync_copy(counts_v, counts_out.at[wid])


def _pixel_body(s_ref, m_ref, w_ref, pt_ref, pv_ref, z_ref,
                sim_out, wl_out, part_out,
                m_v, w_v, buf0, buf1, pt_v, pv_v, nsq_v, dot_v, sim_v, wl_v,
                acc_v, sem0, sem1):
    wid, b, h0 = _worker("c", "s")

    def copy(c0, buf, sem):
        return pltpu.make_async_copy(
            s_ref.at[b, pl.ds(c0 * KCH, KCH), pl.ds(h0, RPW)], buf, sem)

    copy(0, buf0, sem0).start()
    pltpu.sync_copy(m_ref.at[b, pl.ds(h0, RPW)], m_v)
    pltpu.sync_copy(w_ref.at[b, pl.ds(h0, RPW)], w_v)
    pltpu.sync_copy(pt_ref, pt_v)
    pltpu.sync_copy(pv_ref, pv_v)
    pltpu.sync_copy(z_ref, nsq_v)
    pltpu.sync_copy(z_ref, dot_v)
    for k in range(4):
        acc_v[pl.ds(k * L, L)] = jnp.zeros((L,), jnp.float32)

    def compute(c0, buf):
        def r_body(r, _):
            for u in range(GPR):
                off = r * W + u * L
                idx = m_v[r, pl.ds(u * L, L)]
                nsq = nsq_v[pl.ds(off, L)]
                dot = dot_v[pl.ds(off, L)]
                for ch in range(KCH):
                    x = buf[ch, r, pl.ds(u * L, L)]
                    p = plsc.load_gather(
                        pt_v, [idx + (c0 * KCH + ch) * NUM_CLASSES])
                    nsq = nsq + x * x
                    dot = dot + x * p
                nsq_v[pl.ds(off, L)] = nsq
                dot_v[pl.ds(off, L)] = dot
            return 0
        lax.fori_loop(0, RPW, r_body, 0)

    def pair_body(i, _):
        ca = 2 * i
        cb = 2 * i + 1
        copy(ca, buf0, sem0).wait()
        copy(cb, buf1, sem1).start()
        compute(ca, buf0)
        copy(cb, buf1, sem1).wait()

        @pl.when(i < NCHUNK // 2 - 1)
        def _():
            copy(ca + 2, buf0, sem0).start()
        compute(cb, buf1)
        return 0
    lax.fori_loop(0, NCHUNK // 2, pair_body, 0)

    def fin_body(r, _):
        for u in range(GPR):
            off = r * W + u * L
            x = nsq_v[pl.ds(off, L)]
            d = dot_v[pl.ds(off, L)]
            m16 = m_v[r, pl.ds(u * L, L)]
            sw = w_v[r, pl.ds(u * L, L)]
            i = plsc.bitcast(x, jnp.int32)
            i = jnp.int32(0x5F3759DF) - (i >> 1)
            y = plsc.bitcast(i, jnp.float32)
            for _ in range(3):
                y = y * (1.5 - 0.5 * x * y * y)
            nrm = x * y
            sim = jnp.where(nrm > 1e-12, d * y, d * 1e12)
            v = plsc.load_gather(pv_v, [m16])
            wl = (1.0 - sim) * v * sw
            sim_v[r, pl.ds(u * L, L)] = sim
            wl_v[r, pl.ds(u * L, L)] = wl
            acc_v[pl.ds(0, L)] = acc_v[pl.ds(0, L)] + wl
            acc_v[pl.ds(L, L)] = acc_v[pl.ds(L, L)] + v * sw
            acc_v[pl.ds(2 * L, L)] = acc_v[pl.ds(2 * L, L)] + sim * v
            acc_v[pl.ds(3 * L, L)] = acc_v[pl.ds(3 * L, L)] + v
        return 0
    lax.fori_loop(0, RPW, fin_body, 0)

    pltpu.sync_copy(sim_v, sim_out.at[b, pl.ds(h0, RPW)])
    pltpu.sync_copy(wl_v, wl_out.at[b, pl.ds(h0, RPW)])
    pltpu.sync_copy(acc_v, part_out.at[wid])


HBLK = 8
CPAD_TC = 152


def _tc_proto_body(m_ref, t_ref, sums_ref, counts_ref):
    step = pl.program_id(0) * pl.num_programs(1) + pl.program_id(1)

    mrow = m_ref[...].reshape(1, HBLK * W)
    oh = (jax.lax.broadcasted_iota(jnp.int32, (CPAD_TC, HBLK * W), 0)
          == mrow).astype(jnp.float32)
    t = t_ref[...].reshape(FEAT, HBLK * W)
    part = jax.lax.dot_general(t, oh, (((1,), (1,)), ((), ())),
                               precision=jax.lax.Precision.HIGHEST,
                               preferred_element_type=jnp.float32)
    cnt = jnp.sum(oh, axis=1)

    @pl.when(step == 0)
    def _():
        sums_ref[...] = jnp.zeros_like(sums_ref)
        counts_ref[...] = jnp.zeros_like(counts_ref)
    sums_ref[...] += part
    counts_ref[...] += cnt


def _sc_mesh():
    return plsc.VectorSubcoreMesh(core_axis_name="c", subcore_axis_name="s",
                                  num_cores=NC, num_subcores=NS)


_SC_PARAMS = pltpu.CompilerParams(needs_layout_passes=False)


@jax.jit
def kernel(s_feat, t_feat, mask, swt_weight):
    f32 = jnp.float32
    w3 = swt_weight.reshape(B, H, W)
    zeros_px = jnp.zeros((PXW,), f32)

    sums_tc, counts_tc = pl.pallas_call(
        _tc_proto_body,
        grid=(B, H // HBLK),
        in_specs=[
            pl.BlockSpec((1, HBLK, W), lambda b, h: (b, h, 0)),
            pl.BlockSpec((1, FEAT, HBLK, W), lambda b, h: (b, 0, h, 0)),
        ],
        out_specs=[
            pl.BlockSpec((FEAT, CPAD_TC), lambda b, h: (0, 0)),
            pl.BlockSpec((1, CPAD_TC), lambda b, h: (0, 0)),
        ],
        out_shape=(jax.ShapeDtypeStruct((FEAT, CPAD_TC), f32),
                   jax.ShapeDtypeStruct((1, CPAD_TC), f32)),
    )(mask, t_feat)
    sums = sums_tc.T[:NUM_CLASSES]
    counts = counts_tc[0, :NUM_CLASSES]
    means = jnp.where(counts[:, None] > 0,
                      sums / jnp.maximum(counts[:, None], 1.0), 0.0)
    nrm = jnp.linalg.norm(means, axis=-1, keepdims=True)
    protos = means / jnp.maximum(nrm, 1e-6)
    proto_norm = jnp.linalg.norm(protos, axis=-1)
    proto_valid = proto_norm > 0
    pt_flat = protos.T.reshape(-1)
    pv_f = jnp.zeros((CPAD,), f32).at[:NUM_CLASSES].set(proto_valid.astype(f32))

    pixel_call = pl.kernel(
        _pixel_body,
        out_type=(jax.ShapeDtypeStruct((B, H, W), f32),
                  jax.ShapeDtypeStruct((B, H, W), f32),
                  jax.ShapeDtypeStruct((NW, 4 * L), f32)),
        mesh=_sc_mesh(),
        compiler_params=_SC_PARAMS,
        scratch_types=[
            pltpu.VMEM((RPW, W), jnp.int32),
            pltpu.VMEM((RPW, W), f32),
            pltpu.VMEM((KCH, RPW, W), f32),
            pltpu.VMEM((KCH, RPW, W), f32),
            pltpu.VMEM((SUMS,), f32),
            pltpu.VMEM((CPAD,), f32),
            pltpu.VMEM((PXW,), f32),
            pltpu.VMEM((PXW,), f32),
            pltpu.VMEM((RPW, W), f32),
            pltpu.VMEM((RPW, W), f32),
            pltpu.VMEM((4 * L,), f32),
            pltpu.SemaphoreType.DMA,
            pltpu.SemaphoreType.DMA,
        ],
    )
    sim_map, weighted_loss_map, partials = pixel_call(
        s_feat, mask, w3, pt_flat, pv_f, zeros_px)

    p = partials.reshape(NW, 4, L).sum(axis=(0, 2))
    final_loss = p[0] / jnp.maximum(p[1], 1.0)
    mean_sim = p[2] / jnp.maximum(p[3], 1.0)
    return (final_loss, sim_map, weighted_loss_map, mean_sim, proto_norm, proto_valid)
